# Initial kernel scaffold; baseline (speedup 1.0000x reference)
import jax
import jax.numpy as jnp
from jax.experimental import pallas as pl


def kernel(x, edge_index, W1, b1, W2, b2, Wg2, bg2, Wg3, bg3, Wl, bl):
    raise NotImplementedError("write your pallas kernel here")


# speedup vs baseline: 6.2111x; 6.2111x over previous
import functools

import jax
import jax.numpy as jnp
from jax import lax
from jax.experimental import pallas as pl
from jax.experimental.pallas import tpu as pltpu
from jax.experimental.pallas import tpu_sc as plsc

N = 10000
E = 320000
D = 128
H = 128
O = 64

NC = 2
NS = 16
NW = NC * NS

CHUNK = 128
CPT = 80
EPT = CPT * CHUNK
E_PAD = EPT * NW
NACC = 10240
RPT = NACC // NS
IDXB = 16
DUMMY_DST = N


def _sc_scatter_body(with_deg, *refs):
    if with_deg:
        (table, src2d, dst2d, out, deg_out,
         src_v, dst_v, buf_a, buf_b, zb, acc, sem_a, sem_b,
         ones_v, degs_v, dacc) = refs
    else:
        (table, src2d, dst2d, out,
         src_v, dst_v, buf_a, buf_b, zb, acc, sem_a, sem_b) = refs

    c = lax.axis_index("c")
    s = lax.axis_index("s")
    wid = s * NC + c

    for r in range(16):
        for cc in range(H // 16):
            zb[r, pl.ds(cc * 16, 16)] = jnp.zeros((16,), jnp.float32)

    def zero_body(i, carry):
        pltpu.sync_copy(zb, acc.at[pl.ds(s * RPT + i * 16, 16)])
        return carry
    lax.fori_loop(0, RPT // 16, zero_body, 0)

    if with_deg:
        for cc in range(CHUNK // 16):
            ones_v[pl.ds(cc * 16, 16)] = jnp.ones((16,), jnp.float32)

        def dzero_body(i, carry):
            degs_v[pl.ds(i * 16, 16)] = jnp.zeros((16,), jnp.float32)
            return carry
        lax.fori_loop(0, RPT // 16, dzero_body, 0)
        pltpu.sync_copy(degs_v, dacc.at[pl.ds(s * RPT, RPT)])

    plsc.subcore_barrier()

    def blk_body(b, carry):
        pltpu.sync_copy(src2d.at[pl.ds(wid * CPT + b * IDXB, IDXB)], src_v)
        pltpu.sync_copy(dst2d.at[pl.ds(wid * CPT + b * IDXB, IDXB)], dst_v)

        def body(i, carry2):
            j0 = 2 * i
            j1 = 2 * i + 1
            cp_a = pltpu.async_copy(table.at[src_v.at[j0]], buf_a, sem_a)
            cp_b = pltpu.async_copy(table.at[src_v.at[j1]], buf_b, sem_b)
            cp_a.wait()
            pltpu.sync_copy(buf_a, acc.at[dst_v.at[j0]], add=True)
            if with_deg:
                pltpu.sync_copy(ones_v, dacc.at[dst_v.at[j0]], add=True)
            cp_b.wait()
            pltpu.sync_copy(buf_b, acc.at[dst_v.at[j1]], add=True)
            if with_deg:
                pltpu.sync_copy(ones_v, dacc.at[dst_v.at[j1]], add=True)
            return carry2
        lax.fori_loop(0, IDXB // 2, body, 0)
        return carry
    lax.fori_loop(0, CPT // IDXB, blk_body, 0)

    plsc.subcore_barrier()

    def wb_body(k, carry):
        r0 = s * RPT + k * CHUNK
        pltpu.sync_copy(acc.at[pl.ds(r0, CHUNK)], buf_a)
        pltpu.sync_copy(buf_a, out.at[c, pl.ds(r0, CHUNK)])
        return carry
    lax.fori_loop(0, RPT // CHUNK, wb_body, 0)

    if with_deg:
        pltpu.sync_copy(dacc.at[pl.ds(s * RPT, RPT)], degs_v)
        pltpu.sync_copy(degs_v, deg_out.at[c, pl.ds(s * RPT, RPT)])


def _make_sc_scatter(with_deg):
    out_type = [jax.ShapeDtypeStruct((NC, NACC, H), jnp.float32)]
    if with_deg:
        out_type.append(jax.ShapeDtypeStruct((NC, NACC), jnp.float32))
    scratch = [
        pltpu.VMEM((IDXB, CHUNK), jnp.int32),
        pltpu.VMEM((IDXB, CHUNK), jnp.int32),
        pltpu.VMEM((CHUNK, H), jnp.float32),
        pltpu.VMEM((CHUNK, H), jnp.float32),
        pltpu.VMEM((16, H), jnp.float32),
        pltpu.VMEM_SHARED((NACC, H), jnp.float32),
        pltpu.SemaphoreType.DMA,
        pltpu.SemaphoreType.DMA,
    ]
    if with_deg:
        scratch += [
            pltpu.VMEM((CHUNK,), jnp.float32),
            pltpu.VMEM((RPT,), jnp.float32),
            pltpu.VMEM_SHARED((NACC,), jnp.float32),
        ]
    mesh = plsc.VectorSubcoreMesh(core_axis_name="c", subcore_axis_name="s",
                                  num_cores=NC, num_subcores=NS)
    return pl.kernel(
        functools.partial(_sc_scatter_body, with_deg),
        out_type=tuple(out_type) if with_deg else out_type[0],
        mesh=mesh,
        scratch_types=scratch,
    )


_SC_CACHE = {}


def _sc_scatter_deg(table, src_p, dst_p):
    if True not in _SC_CACHE:
        _SC_CACHE[True] = _make_sc_scatter(True)
    return _SC_CACHE[True](table, src_p, dst_p)


def _sc_scatter(table, src_p, dst_p):
    if False not in _SC_CACHE:
        _SC_CACHE[False] = _make_sc_scatter(False)
    return _SC_CACHE[False](table, src_p, dst_p)


BLK = 512
GRID = (NACC + BLK - 1) // BLK


def _dinv(pd_ref):
    deg = pd_ref[0] + pd_ref[1] + 1.0
    return lax.rsqrt(deg)


def _tc_a_body(p_ref, pd_ref, x_ref, w1_ref, b1_ref, w2_ref, b2_ref,
               wg2_ref, y2_ref):
    agg = p_ref[0] + p_ref[1] + x_ref[...]
    h = jnp.maximum(
        jnp.dot(agg, w1_ref[...], preferred_element_type=jnp.float32)
        + b1_ref[...], 0.0)
    h = jnp.dot(h, w2_ref[...], preferred_element_type=jnp.float32) \
        + b2_ref[...]
    hr = jnp.maximum(h, 0.0)
    t2 = jnp.dot(hr, wg2_ref[...], preferred_element_type=jnp.float32)
    y2_ref[...] = _dinv(pd_ref)[:, None] * t2


def _tc_b_body(p_ref, pd_ref, y2_ref, bg2_ref, wg3_ref, y3_ref):
    dinv = _dinv(pd_ref)[:, None]
    out2 = dinv * (p_ref[0] + p_ref[1] + y2_ref[...]) + bg2_ref[...]
    t3 = jnp.dot(out2, wg3_ref[...], preferred_element_type=jnp.float32)
    y3_ref[...] = dinv * t3


def _tc_c_body(p_ref, pd_ref, y3_ref, bg3_ref, wl_ref, bl_ref,
               lsm_ref, emb_ref):
    dinv = _dinv(pd_ref)[:, None]
    emb = dinv * (p_ref[0] + p_ref[1] + y3_ref[...]) + bg3_ref[...]
    logits = jnp.dot(emb, wl_ref[...], preferred_element_type=jnp.float32) \
        + bl_ref[...]
    m = jnp.max(logits, axis=1, keepdims=True)
    z = logits - m
    lse = jnp.log(jnp.sum(jnp.exp(z), axis=1, keepdims=True))
    lsm_ref[...] = z - lse
    emb_ref[...] = emb


def _row_spec(cols):
    return pl.BlockSpec((BLK, cols), lambda i: (i, 0))


_P_SPEC = pl.BlockSpec((2, BLK, H), lambda i: (0, i, 0))
_PD_SPEC = pl.BlockSpec((2, BLK), lambda i: (0, i))


def _w_spec(r, c):
    return pl.BlockSpec((r, c), lambda i: (0, 0))


_tc_a = pl.pallas_call(
    _tc_a_body,
    grid=(GRID,),
    in_specs=[_P_SPEC, _PD_SPEC, _row_spec(D), _w_spec(D, H), _w_spec(1, H),
              _w_spec(H, H), _w_spec(1, H), _w_spec(H, H)],
    out_specs=_row_spec(H),
    out_shape=jax.ShapeDtypeStruct((N, H), jnp.float32),
)

_tc_b = pl.pallas_call(
    _tc_b_body,
    grid=(GRID,),
    in_specs=[_P_SPEC, _PD_SPEC, _row_spec(H), _w_spec(1, H), _w_spec(H, H)],
    out_specs=_row_spec(H),
    out_shape=jax.ShapeDtypeStruct((N, H), jnp.float32),
)

_tc_c = pl.pallas_call(
    _tc_c_body,
    grid=(GRID,),
    in_specs=[_P_SPEC, _PD_SPEC, _row_spec(H), _w_spec(1, H), _w_spec(H, O),
              _w_spec(1, O)],
    out_specs=(_row_spec(O), _row_spec(H)),
    out_shape=(jax.ShapeDtypeStruct((N, O), jnp.float32),
               jax.ShapeDtypeStruct((N, H), jnp.float32)),
)


def kernel(x, edge_index, W1, b1, W2, b2, Wg2, bg2, Wg3, bg3, Wl, bl):
    src = edge_index[0].astype(jnp.int32)
    dst = edge_index[1].astype(jnp.int32)
    npad = E_PAD - E
    src_p = jnp.concatenate(
        [src, jnp.zeros((npad,), jnp.int32)]).reshape(E_PAD // CHUNK, CHUNK)
    dst_p = jnp.concatenate(
        [dst, jnp.full((npad,), DUMMY_DST, jnp.int32)]
    ).reshape(E_PAD // CHUNK, CHUNK)

    b1r = b1.reshape(1, H)
    b2r = b2.reshape(1, H)
    bg2r = bg2.reshape(1, H)
    bg3r = bg3.reshape(1, H)
    blr = bl.reshape(1, O)

    p1, pdeg = _sc_scatter_deg(x, src_p, dst_p)
    y2 = _tc_a(p1, pdeg, x, W1, b1r, W2, b2r, Wg2)
    p2 = _sc_scatter(y2, src_p, dst_p)
    y3 = _tc_b(p2, pdeg, y2, bg2r, Wg3)
    p3 = _sc_scatter(y3, src_p, dst_p)
    lsm, emb = _tc_c(p3, pdeg, y3, bg3r, Wl, blr)
    return (lsm, emb)


# speedup vs baseline: 6.5034x; 1.0471x over previous
import functools

import jax
import jax.numpy as jnp
from jax import lax
from jax.experimental import pallas as pl
from jax.experimental.pallas import tpu as pltpu
from jax.experimental.pallas import tpu_sc as plsc

N = 10000
E = 320000
D = 128
H = 128
O = 64

NC = 2
NS = 16
NW = NC * NS

CHUNK = 128
CPT = 80
EPT = CPT * CHUNK
E_PAD = EPT * NW
NACC = 10240
RPT = NACC // NS
IDXB = 16
DUMMY_DST = N


def _sc_scatter_body(with_deg, *refs):
    if with_deg:
        (table, src2d, dst2d, out, deg_out,
         src_v, dst_v, buf_a, buf_b, zb, acc,
         sem_a0, sem_a1, sem_b0, sem_b1, sem_sa, sem_sb,
         ones_v, degs_v, dacc) = refs
    else:
        (table, src2d, dst2d, out,
         src_v, dst_v, buf_a, buf_b, zb, acc,
         sem_a0, sem_a1, sem_b0, sem_b1, sem_sa, sem_sb) = refs

    c = lax.axis_index("c")
    s = lax.axis_index("s")
    wid = s * NC + c

    for r in range(16):
        for cc in range(H // 16):
            zb[r, pl.ds(cc * 16, 16)] = jnp.zeros((16,), jnp.float32)

    def zero_body(i, carry):
        pltpu.sync_copy(zb, acc.at[pl.ds(s * RPT + i * 16, 16)])
        return carry
    lax.fori_loop(0, RPT // 16, zero_body, 0)

    if with_deg:
        for cc in range(CHUNK // 16):
            ones_v[pl.ds(cc * 16, 16)] = jnp.ones((16,), jnp.float32)

        def dzero_body(i, carry):
            degs_v[pl.ds(i * 16, 16)] = jnp.zeros((16,), jnp.float32)
            return carry
        lax.fori_loop(0, RPT // 16, dzero_body, 0)
        pltpu.sync_copy(degs_v, dacc.at[pl.ds(s * RPT, RPT)])

    plsc.subcore_barrier()

    HALF = CHUNK // 2

    def gather(j, buf, s0, s1):
        pltpu.async_copy(table.at[src_v.at[j, pl.ds(0, HALF)]],
                         buf.at[pl.ds(0, HALF)], s0)
        pltpu.async_copy(table.at[src_v.at[j, pl.ds(HALF, HALF)]],
                         buf.at[pl.ds(HALF, HALF)], s1)

    def wait_gather(j, buf, s0, s1):
        pltpu.make_async_copy(table.at[src_v.at[j, pl.ds(0, HALF)]],
                              buf.at[pl.ds(0, HALF)], s0).wait()
        pltpu.make_async_copy(table.at[src_v.at[j, pl.ds(HALF, HALF)]],
                              buf.at[pl.ds(HALF, HALF)], s1).wait()

    def blk_body(b, carry):
        pltpu.sync_copy(src2d.at[pl.ds(wid * CPT + b * IDXB, IDXB)], src_v)
        pltpu.sync_copy(dst2d.at[pl.ds(wid * CPT + b * IDXB, IDXB)], dst_v)
        gather(0, buf_a, sem_a0, sem_a1)
        gather(1, buf_b, sem_b0, sem_b1)

        def body(i, carry2):
            j0 = 2 * i
            j1 = 2 * i + 1
            wait_gather(j0, buf_a, sem_a0, sem_a1)
            pltpu.async_copy(buf_a, acc.at[dst_v.at[j0]], sem_sa, add=True)
            if with_deg:
                pltpu.sync_copy(ones_v, dacc.at[dst_v.at[j0]], add=True)
            wait_gather(j1, buf_b, sem_b0, sem_b1)
            pltpu.async_copy(buf_b, acc.at[dst_v.at[j1]], sem_sb, add=True)
            if with_deg:
                pltpu.sync_copy(ones_v, dacc.at[dst_v.at[j1]], add=True)

            @pl.when(i < IDXB // 2 - 1)
            def _refill():
                pltpu.make_async_copy(buf_a, acc.at[dst_v.at[j0]],
                                      sem_sa).wait()
                gather(j0 + 2, buf_a, sem_a0, sem_a1)
                pltpu.make_async_copy(buf_b, acc.at[dst_v.at[j1]],
                                      sem_sb).wait()
                gather(j1 + 2, buf_b, sem_b0, sem_b1)
            return carry2
        lax.fori_loop(0, IDXB // 2, body, 0)
        pltpu.make_async_copy(buf_a, acc.at[dst_v.at[IDXB - 2]],
                              sem_sa).wait()
        pltpu.make_async_copy(buf_b, acc.at[dst_v.at[IDXB - 1]],
                              sem_sb).wait()
        return carry
    lax.fori_loop(0, CPT // IDXB, blk_body, 0)

    plsc.subcore_barrier()

    def wb_body(k, carry):
        r0 = s * RPT + k * CHUNK
        pltpu.sync_copy(acc.at[pl.ds(r0, CHUNK)], buf_a)
        pltpu.sync_copy(buf_a, out.at[c, pl.ds(r0, CHUNK)])
        return carry
    lax.fori_loop(0, RPT // CHUNK, wb_body, 0)

    if with_deg:
        pltpu.sync_copy(dacc.at[pl.ds(s * RPT, RPT)], degs_v)
        pltpu.sync_copy(degs_v, deg_out.at[c, pl.ds(s * RPT, RPT)])


def _make_sc_scatter(with_deg):
    out_type = [jax.ShapeDtypeStruct((NC, NACC, H), jnp.float32)]
    if with_deg:
        out_type.append(jax.ShapeDtypeStruct((NC, NACC), jnp.float32))
    scratch = [
        pltpu.VMEM((IDXB, CHUNK), jnp.int32),
        pltpu.VMEM((IDXB, CHUNK), jnp.int32),
        pltpu.VMEM((CHUNK, H), jnp.float32),
        pltpu.VMEM((CHUNK, H), jnp.float32),
        pltpu.VMEM((16, H), jnp.float32),
        pltpu.VMEM_SHARED((NACC, H), jnp.float32),
        pltpu.SemaphoreType.DMA,
        pltpu.SemaphoreType.DMA,
        pltpu.SemaphoreType.DMA,
        pltpu.SemaphoreType.DMA,
        pltpu.SemaphoreType.DMA,
        pltpu.SemaphoreType.DMA,
    ]
    if with_deg:
        scratch += [
            pltpu.VMEM((CHUNK,), jnp.float32),
            pltpu.VMEM((RPT,), jnp.float32),
            pltpu.VMEM_SHARED((NACC,), jnp.float32),
        ]
    mesh = plsc.VectorSubcoreMesh(core_axis_name="c", subcore_axis_name="s",
                                  num_cores=NC, num_subcores=NS)
    return pl.kernel(
        functools.partial(_sc_scatter_body, with_deg),
        out_type=tuple(out_type) if with_deg else out_type[0],
        mesh=mesh,
        scratch_types=scratch,
    )


_SC_CACHE = {}


def _sc_scatter_deg(table, src_p, dst_p):
    if True not in _SC_CACHE:
        _SC_CACHE[True] = _make_sc_scatter(True)
    return _SC_CACHE[True](table, src_p, dst_p)


def _sc_scatter(table, src_p, dst_p):
    if False not in _SC_CACHE:
        _SC_CACHE[False] = _make_sc_scatter(False)
    return _SC_CACHE[False](table, src_p, dst_p)


BLK = 512
GRID = (NACC + BLK - 1) // BLK


def _dinv(pd_ref):
    deg = pd_ref[0] + pd_ref[1] + 1.0
    return lax.rsqrt(deg)


def _tc_a_body(p_ref, pd_ref, x_ref, w1_ref, b1_ref, w2_ref, b2_ref,
               wg2_ref, y2_ref):
    agg = p_ref[0] + p_ref[1] + x_ref[...]
    h = jnp.maximum(
        jnp.dot(agg, w1_ref[...], preferred_element_type=jnp.float32)
        + b1_ref[...], 0.0)
    h = jnp.dot(h, w2_ref[...], preferred_element_type=jnp.float32) \
        + b2_ref[...]
    hr = jnp.maximum(h, 0.0)
    t2 = jnp.dot(hr, wg2_ref[...], preferred_element_type=jnp.float32)
    y2_ref[...] = _dinv(pd_ref)[:, None] * t2


def _tc_b_body(p_ref, pd_ref, y2_ref, bg2_ref, wg3_ref, y3_ref):
    dinv = _dinv(pd_ref)[:, None]
    out2 = dinv * (p_ref[0] + p_ref[1] + y2_ref[...]) + bg2_ref[...]
    t3 = jnp.dot(out2, wg3_ref[...], preferred_element_type=jnp.float32)
    y3_ref[...] = dinv * t3


def _tc_c_body(p_ref, pd_ref, y3_ref, bg3_ref, wl_ref, bl_ref,
               lsm_ref, emb_ref):
    dinv = _dinv(pd_ref)[:, None]
    emb = dinv * (p_ref[0] + p_ref[1] + y3_ref[...]) + bg3_ref[...]
    logits = jnp.dot(emb, wl_ref[...], preferred_element_type=jnp.float32) \
        + bl_ref[...]
    m = jnp.max(logits, axis=1, keepdims=True)
    z = logits - m
    lse = jnp.log(jnp.sum(jnp.exp(z), axis=1, keepdims=True))
    lsm_ref[...] = z - lse
    emb_ref[...] = emb


def _row_spec(cols):
    return pl.BlockSpec((BLK, cols), lambda i: (i, 0))


_P_SPEC = pl.BlockSpec((2, BLK, H), lambda i: (0, i, 0))
_PD_SPEC = pl.BlockSpec((2, BLK), lambda i: (0, i))


def _w_spec(r, c):
    return pl.BlockSpec((r, c), lambda i: (0, 0))


_tc_a = pl.pallas_call(
    _tc_a_body,
    grid=(GRID,),
    in_specs=[_P_SPEC, _PD_SPEC, _row_spec(D), _w_spec(D, H), _w_spec(1, H),
              _w_spec(H, H), _w_spec(1, H), _w_spec(H, H)],
    out_specs=_row_spec(H),
    out_shape=jax.ShapeDtypeStruct((N, H), jnp.float32),
)

_tc_b = pl.pallas_call(
    _tc_b_body,
    grid=(GRID,),
    in_specs=[_P_SPEC, _PD_SPEC, _row_spec(H), _w_spec(1, H), _w_spec(H, H)],
    out_specs=_row_spec(H),
    out_shape=jax.ShapeDtypeStruct((N, H), jnp.float32),
)

_tc_c = pl.pallas_call(
    _tc_c_body,
    grid=(GRID,),
    in_specs=[_P_SPEC, _PD_SPEC, _row_spec(H), _w_spec(1, H), _w_spec(H, O),
              _w_spec(1, O)],
    out_specs=(_row_spec(O), _row_spec(H)),
    out_shape=(jax.ShapeDtypeStruct((N, O), jnp.float32),
               jax.ShapeDtypeStruct((N, H), jnp.float32)),
)


def kernel(x, edge_index, W1, b1, W2, b2, Wg2, bg2, Wg3, bg3, Wl, bl):
    src = edge_index[0].astype(jnp.int32)
    dst = edge_index[1].astype(jnp.int32)
    npad = E_PAD - E
    src_p = jnp.concatenate(
        [src, jnp.zeros((npad,), jnp.int32)]).reshape(E_PAD // CHUNK, CHUNK)
    dst_p = jnp.concatenate(
        [dst, jnp.full((npad,), DUMMY_DST, jnp.int32)]
    ).reshape(E_PAD // CHUNK, CHUNK)

    b1r = b1.reshape(1, H)
    b2r = b2.reshape(1, H)
    bg2r = bg2.reshape(1, H)
    bg3r = bg3.reshape(1, H)
    blr = bl.reshape(1, O)

    p1, pdeg = _sc_scatter_deg(x, src_p, dst_p)
    y2 = _tc_a(p1, pdeg, x, W1, b1r, W2, b2r, Wg2)
    p2 = _sc_scatter(y2, src_p, dst_p)
    y3 = _tc_b(p2, pdeg, y2, bg2r, Wg3)
    p3 = _sc_scatter(y3, src_p, dst_p)
    lsm, emb = _tc_c(p3, pdeg, y3, bg3r, Wl, blr)
    return (lsm, emb)


# speedup vs baseline: 18.8290x; 2.8953x over previous
import functools

import jax
import jax.numpy as jnp
from jax import lax
from jax.experimental import pallas as pl
from jax.experimental.pallas import tpu as pltpu
from jax.experimental.pallas import tpu_sc as plsc

N = 10000
E = 320000
D = 128
H = 128
O = 64

NC = 2
NS = 16
NW = NC * NS

CHUNK = 128
CPT = 80
EPT = CPT * CHUNK
E_PAD = EPT * NW
NACC = 10240
RPT = NACC // NS
IDXB = 16
DUMMY_DST = N


def _sc_scatter_body(with_deg, *refs):
    if with_deg:
        (table, src2d, dst2d, out, deg_out,
         src_v, dst_v, buf_a, buf_b, zb, acc,
         sem_a0, sem_a1, sem_b0, sem_b1, sem_sa, sem_sb,
         ones_v, degs_v, dacc) = refs
    else:
        (table, src2d, dst2d, out,
         src_v, dst_v, buf_a, buf_b, zb, acc,
         sem_a0, sem_a1, sem_b0, sem_b1, sem_sa, sem_sb) = refs

    c = lax.axis_index("c")
    s = lax.axis_index("s")
    wid = s * NC + c

    for r in range(16):
        for cc in range(H // 16):
            zb[r, pl.ds(cc * 16, 16)] = jnp.zeros((16,), jnp.float32)

    def zero_body(i, carry):
        pltpu.sync_copy(zb, acc.at[pl.ds(s * RPT + i * 16, 16)])
        return carry
    lax.fori_loop(0, RPT // 16, zero_body, 0)

    if with_deg:
        for cc in range(CHUNK // 16):
            ones_v[pl.ds(cc * 16, 16)] = jnp.ones((16,), jnp.float32)

        def dzero_body(i, carry):
            degs_v[pl.ds(i * 16, 16)] = jnp.zeros((16,), jnp.float32)
            return carry
        lax.fori_loop(0, RPT // 16, dzero_body, 0)
        pltpu.sync_copy(degs_v, dacc.at[pl.ds(s * RPT, RPT)])

    plsc.subcore_barrier()

    HALF = CHUNK // 2

    def gather(j, buf, s0, s1):
        pltpu.async_copy(table.at[src_v.at[j, pl.ds(0, HALF)]],
                         buf.at[pl.ds(0, HALF)], s0)
        pltpu.async_copy(table.at[src_v.at[j, pl.ds(HALF, HALF)]],
                         buf.at[pl.ds(HALF, HALF)], s1)

    def wait_gather(j, buf, s0, s1):
        pltpu.make_async_copy(table.at[src_v.at[j, pl.ds(0, HALF)]],
                              buf.at[pl.ds(0, HALF)], s0).wait()
        pltpu.make_async_copy(table.at[src_v.at[j, pl.ds(HALF, HALF)]],
                              buf.at[pl.ds(HALF, HALF)], s1).wait()

    def blk_body(b, carry):
        pltpu.sync_copy(src2d.at[pl.ds(wid * CPT + b * IDXB, IDXB)], src_v)
        pltpu.sync_copy(dst2d.at[pl.ds(wid * CPT + b * IDXB, IDXB)], dst_v)
        gather(0, buf_a, sem_a0, sem_a1)
        gather(1, buf_b, sem_b0, sem_b1)

        def body(i, carry2):
            j0 = 2 * i
            j1 = 2 * i + 1
            wait_gather(j0, buf_a, sem_a0, sem_a1)
            pltpu.async_copy(buf_a, acc.at[dst_v.at[j0]], sem_sa, add=True)
            if with_deg:
                pltpu.sync_copy(ones_v, dacc.at[dst_v.at[j0]], add=True)
            wait_gather(j1, buf_b, sem_b0, sem_b1)
            pltpu.async_copy(buf_b, acc.at[dst_v.at[j1]], sem_sb, add=True)
            if with_deg:
                pltpu.sync_copy(ones_v, dacc.at[dst_v.at[j1]], add=True)

            @pl.when(i < IDXB // 2 - 1)
            def _refill():
                pltpu.make_async_copy(buf_a, acc.at[dst_v.at[j0]],
                                      sem_sa).wait()
                gather(j0 + 2, buf_a, sem_a0, sem_a1)
                pltpu.make_async_copy(buf_b, acc.at[dst_v.at[j1]],
                                      sem_sb).wait()
                gather(j1 + 2, buf_b, sem_b0, sem_b1)
            return carry2
        lax.fori_loop(0, IDXB // 2, body, 0)
        pltpu.make_async_copy(buf_a, acc.at[dst_v.at[IDXB - 2]],
                              sem_sa).wait()
        pltpu.make_async_copy(buf_b, acc.at[dst_v.at[IDXB - 1]],
                              sem_sb).wait()
        return carry
    lax.fori_loop(0, CPT // IDXB, blk_body, 0)

    plsc.subcore_barrier()

    def wb_body(k, carry):
        r0 = s * RPT + k * CHUNK
        pltpu.sync_copy(acc.at[pl.ds(r0, CHUNK)], buf_a)
        pltpu.sync_copy(buf_a, out.at[c, pl.ds(r0, CHUNK)])
        return carry
    lax.fori_loop(0, RPT // CHUNK, wb_body, 0)

    if with_deg:
        pltpu.sync_copy(dacc.at[pl.ds(s * RPT, RPT)], degs_v)
        pltpu.sync_copy(degs_v, deg_out.at[c, pl.ds(s * RPT, RPT)])


def _make_sc_scatter(with_deg):
    out_type = [jax.ShapeDtypeStruct((NC, NACC, H), jnp.float32)]
    if with_deg:
        out_type.append(jax.ShapeDtypeStruct((NC, NACC), jnp.float32))
    scratch = [
        pltpu.VMEM((IDXB, CHUNK), jnp.int32),
        pltpu.VMEM((IDXB, CHUNK), jnp.int32),
        pltpu.VMEM((CHUNK, H), jnp.float32),
        pltpu.VMEM((CHUNK, H), jnp.float32),
        pltpu.VMEM((16, H), jnp.float32),
        pltpu.VMEM_SHARED((NACC, H), jnp.float32),
        pltpu.SemaphoreType.DMA,
        pltpu.SemaphoreType.DMA,
        pltpu.SemaphoreType.DMA,
        pltpu.SemaphoreType.DMA,
        pltpu.SemaphoreType.DMA,
        pltpu.SemaphoreType.DMA,
    ]
    if with_deg:
        scratch += [
            pltpu.VMEM((CHUNK,), jnp.float32),
            pltpu.VMEM((RPT,), jnp.float32),
            pltpu.VMEM_SHARED((NACC,), jnp.float32),
        ]
    mesh = plsc.VectorSubcoreMesh(core_axis_name="c", subcore_axis_name="s",
                                  num_cores=NC, num_subcores=NS)
    return pl.kernel(
        functools.partial(_sc_scatter_body, with_deg),
        out_type=tuple(out_type) if with_deg else out_type[0],
        mesh=mesh,
        scratch_types=scratch,
    )


_SC_CACHE = {}


def _sc_scatter_deg(table, src_p, dst_p):
    if True not in _SC_CACHE:
        _SC_CACHE[True] = _make_sc_scatter(True)
    return _SC_CACHE[True](table, src_p, dst_p)


def _sc_scatter(table, src_p, dst_p):
    if False not in _SC_CACHE:
        _SC_CACHE[False] = _make_sc_scatter(False)
    return _SC_CACHE[False](table, src_p, dst_p)


BLK = 512
GRID = (NACC + BLK - 1) // BLK


def _dinv(pd_ref):
    deg = pd_ref[0] + pd_ref[1] + 1.0
    return lax.rsqrt(deg)


def _tc_a_body(p_ref, pd_ref, x_ref, w1_ref, b1_ref, w2_ref, b2_ref,
               wg2_ref, y2_ref):
    agg = p_ref[0] + p_ref[1] + x_ref[...]
    h = jnp.maximum(
        jnp.dot(agg, w1_ref[...], preferred_element_type=jnp.float32)
        + b1_ref[...], 0.0)
    h = jnp.dot(h, w2_ref[...], preferred_element_type=jnp.float32) \
        + b2_ref[...]
    hr = jnp.maximum(h, 0.0)
    t2 = jnp.dot(hr, wg2_ref[...], preferred_element_type=jnp.float32)
    y2_ref[...] = _dinv(pd_ref)[:, None] * t2


def _tc_b_body(p_ref, pd_ref, y2_ref, bg2_ref, wg3_ref, y3_ref):
    dinv = _dinv(pd_ref)[:, None]
    out2 = dinv * (p_ref[0] + p_ref[1] + y2_ref[...]) + bg2_ref[...]
    t3 = jnp.dot(out2, wg3_ref[...], preferred_element_type=jnp.float32)
    y3_ref[...] = dinv * t3


def _tc_c_body(p_ref, pd_ref, y3_ref, bg3_ref, wl_ref, bl_ref,
               lsm_ref, emb_ref):
    dinv = _dinv(pd_ref)[:, None]
    emb = dinv * (p_ref[0] + p_ref[1] + y3_ref[...]) + bg3_ref[...]
    logits = jnp.dot(emb, wl_ref[...], preferred_element_type=jnp.float32) \
        + bl_ref[...]
    m = jnp.max(logits, axis=1, keepdims=True)
    z = logits - m
    lse = jnp.log(jnp.sum(jnp.exp(z), axis=1, keepdims=True))
    lsm_ref[...] = z - lse
    emb_ref[...] = emb


def _row_spec(cols):
    return pl.BlockSpec((BLK, cols), lambda i: (i, 0))


_P_SPEC = pl.BlockSpec((2, BLK, H), lambda i: (0, i, 0))
_PD_SPEC = pl.BlockSpec((2, BLK), lambda i: (0, i))


def _w_spec(r, c):
    return pl.BlockSpec((r, c), lambda i: (0, 0))


_tc_a = pl.pallas_call(
    _tc_a_body,
    grid=(GRID,),
    in_specs=[_P_SPEC, _PD_SPEC, _row_spec(D), _w_spec(D, H), _w_spec(1, H),
              _w_spec(H, H), _w_spec(1, H), _w_spec(H, H)],
    out_specs=_row_spec(H),
    out_shape=jax.ShapeDtypeStruct((N, H), jnp.float32),
)

_tc_b = pl.pallas_call(
    _tc_b_body,
    grid=(GRID,),
    in_specs=[_P_SPEC, _PD_SPEC, _row_spec(H), _w_spec(1, H), _w_spec(H, H)],
    out_specs=_row_spec(H),
    out_shape=jax.ShapeDtypeStruct((N, H), jnp.float32),
)

_tc_c = pl.pallas_call(
    _tc_c_body,
    grid=(GRID,),
    in_specs=[_P_SPEC, _PD_SPEC, _row_spec(H), _w_spec(1, H), _w_spec(H, O),
              _w_spec(1, O)],
    out_specs=(_row_spec(O), _row_spec(H)),
    out_shape=(jax.ShapeDtypeStruct((N, O), jnp.float32),
               jax.ShapeDtypeStruct((N, H), jnp.float32)),
)


def kernel(x, edge_index, W1, b1, W2, b2, Wg2, bg2, Wg3, bg3, Wl, bl):
    src = edge_index[0].astype(jnp.int32)
    dst = edge_index[1].astype(jnp.int32)
    npad = E_PAD - E
    pad_iota = lax.iota(jnp.int32, npad)
    src_p = jnp.concatenate(
        [src, pad_iota % N]).reshape(E_PAD // CHUNK, CHUNK)
    dst_p = jnp.concatenate(
        [dst, N + pad_iota % (NACC - N)]
    ).reshape(E_PAD // CHUNK, CHUNK)

    b1r = b1.reshape(1, H)
    b2r = b2.reshape(1, H)
    bg2r = bg2.reshape(1, H)
    bg3r = bg3.reshape(1, H)
    blr = bl.reshape(1, O)

    p1, pdeg = _sc_scatter_deg(x, src_p, dst_p)
    y2 = _tc_a(p1, pdeg, x, W1, b1r, W2, b2r, Wg2)
    p2 = _sc_scatter(y2, src_p, dst_p)
    y3 = _tc_b(p2, pdeg, y2, bg2r, Wg3)
    p3 = _sc_scatter(y3, src_p, dst_p)
    lsm, emb = _tc_c(p3, pdeg, y3, bg3r, Wl, blr)
    return (lsm, emb)


# speedup vs baseline: 20.3739x; 1.0820x over previous
import functools

import jax
import jax.numpy as jnp
from jax import lax
from jax.experimental import pallas as pl
from jax.experimental.pallas import tpu as pltpu
from jax.experimental.pallas import tpu_sc as plsc

N = 10000
E = 320000
D = 128
H = 128
O = 64

NC = 2
NS = 16
NW = NC * NS

CHUNK = 128
CPT = 80
EPT = CPT * CHUNK
E_PAD = EPT * NW
NACC = 10240
RPT = NACC // NS
IDXB = 16
DUMMY_DST = N


def _sc_scatter_body(with_deg, *refs):
    if with_deg:
        (table, src2d, dst2d, out, deg_out,
         src_v, dst_v, buf_a, buf_b, acc,
         sems_a, sems_b, sem_sa, sem_sb,
         ones_v, degs_v, dacc) = refs
    else:
        (table, src2d, dst2d, out,
         src_v, dst_v, buf_a, buf_b, acc,
         sems_a, sems_b, sem_sa, sem_sb) = refs

    c = lax.axis_index("c")
    s = lax.axis_index("s")
    wid = s * NC + c

    def zfill_body(r, carry):
        for cc in range(H // 16):
            buf_b[r, pl.ds(cc * 16, 16)] = jnp.zeros((16,), jnp.float32)
        return carry
    lax.fori_loop(0, CHUNK, zfill_body, 0)

    def zero_body(i, carry):
        pltpu.sync_copy(buf_b, acc.at[pl.ds(s * RPT + i * CHUNK, CHUNK)])
        return carry
    lax.fori_loop(0, RPT // CHUNK, zero_body, 0)

    if with_deg:
        for cc in range(CHUNK // 16):
            ones_v[pl.ds(cc * 16, 16)] = jnp.ones((16,), jnp.float32)

        def dzero_body(i, carry):
            degs_v[pl.ds(i * 16, 16)] = jnp.zeros((16,), jnp.float32)
            return carry
        lax.fori_loop(0, RPT // 16, dzero_body, 0)
        pltpu.sync_copy(degs_v, dacc.at[pl.ds(s * RPT, RPT)])

    plsc.subcore_barrier()

    QTR = CHUNK // 4

    def gather(j, buf, sems):
        for q in range(4):
            pltpu.async_copy(table.at[src_v.at[j, pl.ds(q * QTR, QTR)]],
                             buf.at[pl.ds(q * QTR, QTR)], sems[q])

    def wait_gather(j, buf, sems):
        for q in range(4):
            pltpu.make_async_copy(table.at[src_v.at[j, pl.ds(q * QTR, QTR)]],
                                  buf.at[pl.ds(q * QTR, QTR)],
                                  sems[q]).wait()

    def blk_body(b, carry):
        pltpu.sync_copy(src2d.at[pl.ds(wid * CPT + b * IDXB, IDXB)], src_v)
        pltpu.sync_copy(dst2d.at[pl.ds(wid * CPT + b * IDXB, IDXB)], dst_v)
        gather(0, buf_a, sems_a)
        gather(1, buf_b, sems_b)

        def body(i, carry2):
            j0 = 2 * i
            j1 = 2 * i + 1
            wait_gather(j0, buf_a, sems_a)
            pltpu.async_copy(buf_a, acc.at[dst_v.at[j0]], sem_sa, add=True)
            if with_deg:
                pltpu.sync_copy(ones_v, dacc.at[dst_v.at[j0]], add=True)
            wait_gather(j1, buf_b, sems_b)
            pltpu.async_copy(buf_b, acc.at[dst_v.at[j1]], sem_sb, add=True)
            if with_deg:
                pltpu.sync_copy(ones_v, dacc.at[dst_v.at[j1]], add=True)

            @pl.when(i < IDXB // 2 - 1)
            def _refill():
                pltpu.make_async_copy(buf_a, acc.at[dst_v.at[j0]],
                                      sem_sa).wait()
                gather(j0 + 2, buf_a, sems_a)
                pltpu.make_async_copy(buf_b, acc.at[dst_v.at[j1]],
                                      sem_sb).wait()
                gather(j1 + 2, buf_b, sems_b)
            return carry2
        lax.fori_loop(0, IDXB // 2, body, 0)
        pltpu.make_async_copy(buf_a, acc.at[dst_v.at[IDXB - 2]],
                              sem_sa).wait()
        pltpu.make_async_copy(buf_b, acc.at[dst_v.at[IDXB - 1]],
                              sem_sb).wait()
        return carry
    lax.fori_loop(0, CPT // IDXB, blk_body, 0)

    plsc.subcore_barrier()

    pending = [None, None]
    for k in range(RPT // CHUNK):
        r0 = s * RPT + k * CHUNK
        buf, sem = (buf_a, sem_sa) if k % 2 == 0 else (buf_b, sem_sb)
        if pending[k % 2] is not None:
            pending[k % 2].wait()
        pltpu.sync_copy(acc.at[pl.ds(r0, CHUNK)], buf)
        pending[k % 2] = pltpu.async_copy(buf, out.at[c, pl.ds(r0, CHUNK)],
                                          sem)
    for p in pending:
        if p is not None:
            p.wait()

    if with_deg:
        pltpu.sync_copy(dacc.at[pl.ds(s * RPT, RPT)], degs_v)
        pltpu.sync_copy(degs_v, deg_out.at[c, pl.ds(s * RPT, RPT)])


def _make_sc_scatter(with_deg):
    out_type = [jax.ShapeDtypeStruct((NC, NACC, H), jnp.float32)]
    if with_deg:
        out_type.append(jax.ShapeDtypeStruct((NC, NACC), jnp.float32))
    scratch = [
        pltpu.VMEM((IDXB, CHUNK), jnp.int32),
        pltpu.VMEM((IDXB, CHUNK), jnp.int32),
        pltpu.VMEM((CHUNK, H), jnp.float32),
        pltpu.VMEM((CHUNK, H), jnp.float32),
        pltpu.VMEM_SHARED((NACC, H), jnp.float32),
        [pltpu.SemaphoreType.DMA] * 4,
        [pltpu.SemaphoreType.DMA] * 4,
        pltpu.SemaphoreType.DMA,
        pltpu.SemaphoreType.DMA,
    ]
    if with_deg:
        scratch += [
            pltpu.VMEM((CHUNK,), jnp.float32),
            pltpu.VMEM((RPT,), jnp.float32),
            pltpu.VMEM_SHARED((NACC,), jnp.float32),
        ]
    mesh = plsc.VectorSubcoreMesh(core_axis_name="c", subcore_axis_name="s",
                                  num_cores=NC, num_subcores=NS)
    return pl.kernel(
        functools.partial(_sc_scatter_body, with_deg),
        out_type=tuple(out_type) if with_deg else out_type[0],
        mesh=mesh,
        scratch_types=scratch,
    )


_SC_CACHE = {}


def _sc_scatter_deg(table, src_p, dst_p):
    if True not in _SC_CACHE:
        _SC_CACHE[True] = _make_sc_scatter(True)
    return _SC_CACHE[True](table, src_p, dst_p)


def _sc_scatter(table, src_p, dst_p):
    if False not in _SC_CACHE:
        _SC_CACHE[False] = _make_sc_scatter(False)
    return _SC_CACHE[False](table, src_p, dst_p)


BLK = 512
GRID = (NACC + BLK - 1) // BLK


def _dinv(pd_ref):
    deg = pd_ref[0] + pd_ref[1] + 1.0
    return lax.rsqrt(deg)


def _tc_a_body(p_ref, pd_ref, x_ref, w1_ref, b1_ref, w2_ref, b2_ref,
               wg2_ref, y2_ref):
    agg = p_ref[0] + p_ref[1] + x_ref[...]
    h = jnp.maximum(
        jnp.dot(agg, w1_ref[...], preferred_element_type=jnp.float32)
        + b1_ref[...], 0.0)
    h = jnp.dot(h, w2_ref[...], preferred_element_type=jnp.float32) \
        + b2_ref[...]
    hr = jnp.maximum(h, 0.0)
    t2 = jnp.dot(hr, wg2_ref[...], preferred_element_type=jnp.float32)
    y2_ref[...] = _dinv(pd_ref)[:, None] * t2


def _tc_b_body(p_ref, pd_ref, y2_ref, bg2_ref, wg3_ref, y3_ref):
    dinv = _dinv(pd_ref)[:, None]
    out2 = dinv * (p_ref[0] + p_ref[1] + y2_ref[...]) + bg2_ref[...]
    t3 = jnp.dot(out2, wg3_ref[...], preferred_element_type=jnp.float32)
    y3_ref[...] = dinv * t3


def _tc_c_body(p_ref, pd_ref, y3_ref, bg3_ref, wl_ref, bl_ref,
               lsm_ref, emb_ref):
    dinv = _dinv(pd_ref)[:, None]
    emb = dinv * (p_ref[0] + p_ref[1] + y3_ref[...]) + bg3_ref[...]
    logits = jnp.dot(emb, wl_ref[...], preferred_element_type=jnp.float32) \
        + bl_ref[...]
    m = jnp.max(logits, axis=1, keepdims=True)
    z = logits - m
    lse = jnp.log(jnp.sum(jnp.exp(z), axis=1, keepdims=True))
    lsm_ref[...] = z - lse
    emb_ref[...] = emb


def _row_spec(cols):
    return pl.BlockSpec((BLK, cols), lambda i: (i, 0))


_P_SPEC = pl.BlockSpec((2, BLK, H), lambda i: (0, i, 0))
_PD_SPEC = pl.BlockSpec((2, BLK), lambda i: (0, i))


def _w_spec(r, c):
    return pl.BlockSpec((r, c), lambda i: (0, 0))


_tc_a = pl.pallas_call(
    _tc_a_body,
    grid=(GRID,),
    in_specs=[_P_SPEC, _PD_SPEC, _row_spec(D), _w_spec(D, H), _w_spec(1, H),
              _w_spec(H, H), _w_spec(1, H), _w_spec(H, H)],
    out_specs=_row_spec(H),
    out_shape=jax.ShapeDtypeStruct((N, H), jnp.float32),
)

_tc_b = pl.pallas_call(
    _tc_b_body,
    grid=(GRID,),
    in_specs=[_P_SPEC, _PD_SPEC, _row_spec(H), _w_spec(1, H), _w_spec(H, H)],
    out_specs=_row_spec(H),
    out_shape=jax.ShapeDtypeStruct((N, H), jnp.float32),
)

_tc_c = pl.pallas_call(
    _tc_c_body,
    grid=(GRID,),
    in_specs=[_P_SPEC, _PD_SPEC, _row_spec(H), _w_spec(1, H), _w_spec(H, O),
              _w_spec(1, O)],
    out_specs=(_row_spec(O), _row_spec(H)),
    out_shape=(jax.ShapeDtypeStruct((N, O), jnp.float32),
               jax.ShapeDtypeStruct((N, H), jnp.float32)),
)


def kernel(x, edge_index, W1, b1, W2, b2, Wg2, bg2, Wg3, bg3, Wl, bl):
    src = edge_index[0].astype(jnp.int32)
    dst = edge_index[1].astype(jnp.int32)
    npad = E_PAD - E
    pad_iota = lax.iota(jnp.int32, npad)
    src_p = jnp.concatenate(
        [src, pad_iota % N]).reshape(E_PAD // CHUNK, CHUNK)
    dst_p = jnp.concatenate(
        [dst, N + pad_iota % (NACC - N)]
    ).reshape(E_PAD // CHUNK, CHUNK)

    b1r = b1.reshape(1, H)
    b2r = b2.reshape(1, H)
    bg2r = bg2.reshape(1, H)
    bg3r = bg3.reshape(1, H)
    blr = bl.reshape(1, O)

    p1, pdeg = _sc_scatter_deg(x, src_p, dst_p)
    y2 = _tc_a(p1, pdeg, x, W1, b1r, W2, b2r, Wg2)
    p2 = _sc_scatter(y2, src_p, dst_p)
    y3 = _tc_b(p2, pdeg, y2, bg2r, Wg3)
    p3 = _sc_scatter(y3, src_p, dst_p)
    lsm, emb = _tc_c(p3, pdeg, y3, bg3r, Wl, blr)
    return (lsm, emb)


# speedup vs baseline: 20.8333x; 1.0225x over previous
import functools

import jax
import jax.numpy as jnp
from jax import lax
from jax.experimental import pallas as pl
from jax.experimental.pallas import tpu as pltpu
from jax.experimental.pallas import tpu_sc as plsc

N = 10000
E = 320000
D = 128
H = 128
O = 64

NC = 2
NS = 16
NW = NC * NS

CHUNK = 128
CPT = 80
EPT = CPT * CHUNK
E_PAD = EPT * NW
NACC = 10240
RPT = NACC // NS
IDXB = 16
DUMMY_DST = N


def _sc_scatter_body(with_deg, *refs):
    if with_deg:
        (table, src2d, dst2d, out, deg_out,
         src_v, dst_v, buf_a, buf_b, acc,
         sems_a, sems_b, sems_sa, sems_sb,
         ones_v, degs_v, dacc) = refs
    else:
        (table, src2d, dst2d, out,
         src_v, dst_v, buf_a, buf_b, acc,
         sems_a, sems_b, sems_sa, sems_sb) = refs

    c = lax.axis_index("c")
    s = lax.axis_index("s")
    wid = s * NC + c

    def zfill_body(r, carry):
        for cc in range(H // 16):
            buf_b[r, pl.ds(cc * 16, 16)] = jnp.zeros((16,), jnp.float32)
        return carry
    lax.fori_loop(0, CHUNK, zfill_body, 0)

    def zero_body(i, carry):
        pltpu.sync_copy(buf_b, acc.at[pl.ds(s * RPT + i * CHUNK, CHUNK)])
        return carry
    lax.fori_loop(0, RPT // CHUNK, zero_body, 0)

    if with_deg:
        for cc in range(CHUNK // 2 // 16):
            ones_v[pl.ds(cc * 16, 16)] = jnp.ones((16,), jnp.float32)

        def dzero_body(i, carry):
            degs_v[pl.ds(i * 16, 16)] = jnp.zeros((16,), jnp.float32)
            return carry
        lax.fori_loop(0, RPT // 16, dzero_body, 0)
        pltpu.sync_copy(degs_v, dacc.at[pl.ds(s * RPT, RPT)])

    plsc.subcore_barrier()

    QTR = CHUNK // 4

    def gather(j, buf, sems):
        for q in range(4):
            pltpu.async_copy(table.at[src_v.at[j, pl.ds(q * QTR, QTR)]],
                             buf.at[pl.ds(q * QTR, QTR)], sems[q])

    def wait_gather(j, buf, sems):
        for q in range(4):
            pltpu.make_async_copy(table.at[src_v.at[j, pl.ds(q * QTR, QTR)]],
                                  buf.at[pl.ds(q * QTR, QTR)],
                                  sems[q]).wait()

    HALF = CHUNK // 2

    def scatter(j, buf, sems):
        pltpu.async_copy(buf.at[pl.ds(0, HALF)], acc.at[dst_v.at[2 * j]],
                         sems[0], add=True)
        pltpu.async_copy(buf.at[pl.ds(HALF, HALF)],
                         acc.at[dst_v.at[2 * j + 1]], sems[1], add=True)

    def wait_scatter(j, buf, sems):
        pltpu.make_async_copy(buf.at[pl.ds(0, HALF)],
                              acc.at[dst_v.at[2 * j]], sems[0]).wait()
        pltpu.make_async_copy(buf.at[pl.ds(HALF, HALF)],
                              acc.at[dst_v.at[2 * j + 1]], sems[1]).wait()

    def deg_scatter(j):
        pltpu.sync_copy(ones_v, dacc.at[dst_v.at[2 * j]], add=True)
        pltpu.sync_copy(ones_v, dacc.at[dst_v.at[2 * j + 1]], add=True)

    def blk_body(b, carry):
        pltpu.sync_copy(src2d.at[pl.ds(wid * CPT + b * IDXB, IDXB)], src_v)
        pltpu.sync_copy(
            dst2d.at[pl.ds(2 * (wid * CPT + b * IDXB), 2 * IDXB)], dst_v)
        gather(0, buf_a, sems_a)
        gather(1, buf_b, sems_b)

        def body(i, carry2):
            j0 = 2 * i
            j1 = 2 * i + 1
            wait_gather(j0, buf_a, sems_a)
            scatter(j0, buf_a, sems_sa)
            if with_deg:
                deg_scatter(j0)
            wait_gather(j1, buf_b, sems_b)
            scatter(j1, buf_b, sems_sb)
            if with_deg:
                deg_scatter(j1)

            @pl.when(i < IDXB // 2 - 1)
            def _refill():
                wait_scatter(j0, buf_a, sems_sa)
                gather(j0 + 2, buf_a, sems_a)
                wait_scatter(j1, buf_b, sems_sb)
                gather(j1 + 2, buf_b, sems_b)
            return carry2
        lax.fori_loop(0, IDXB // 2, body, 0)
        wait_scatter(IDXB - 2, buf_a, sems_sa)
        wait_scatter(IDXB - 1, buf_b, sems_sb)
        return carry
    lax.fori_loop(0, CPT // IDXB, blk_body, 0)

    plsc.subcore_barrier()

    pending = [None, None]
    for k in range(RPT // CHUNK):
        r0 = s * RPT + k * CHUNK
        buf, sem = (buf_a, sems_sa[0]) if k % 2 == 0 else (buf_b, sems_sb[0])
        if pending[k % 2] is not None:
            pending[k % 2].wait()
        pltpu.sync_copy(acc.at[pl.ds(r0, CHUNK)], buf)
        pending[k % 2] = pltpu.async_copy(buf, out.at[c, pl.ds(r0, CHUNK)],
                                          sem)
    for p in pending:
        if p is not None:
            p.wait()

    if with_deg:
        pltpu.sync_copy(dacc.at[pl.ds(s * RPT, RPT)], degs_v)
        pltpu.sync_copy(degs_v, deg_out.at[c, pl.ds(s * RPT, RPT)])


def _make_sc_scatter(with_deg):
    out_type = [jax.ShapeDtypeStruct((NC, NACC, H), jnp.float32)]
    if with_deg:
        out_type.append(jax.ShapeDtypeStruct((NC, NACC), jnp.float32))
    scratch = [
        pltpu.VMEM((IDXB, CHUNK), jnp.int32),
        pltpu.VMEM((2 * IDXB, CHUNK // 2), jnp.int32),
        pltpu.VMEM((CHUNK, H), jnp.float32),
        pltpu.VMEM((CHUNK, H), jnp.float32),
        pltpu.VMEM_SHARED((NACC, H), jnp.float32),
        [pltpu.SemaphoreType.DMA] * 4,
        [pltpu.SemaphoreType.DMA] * 4,
        [pltpu.SemaphoreType.DMA] * 2,
        [pltpu.SemaphoreType.DMA] * 2,
    ]
    if with_deg:
        scratch += [
            pltpu.VMEM((CHUNK // 2,), jnp.float32),
            pltpu.VMEM((RPT,), jnp.float32),
            pltpu.VMEM_SHARED((NACC,), jnp.float32),
        ]
    mesh = plsc.VectorSubcoreMesh(core_axis_name="c", subcore_axis_name="s",
                                  num_cores=NC, num_subcores=NS)
    return pl.kernel(
        functools.partial(_sc_scatter_body, with_deg),
        out_type=tuple(out_type) if with_deg else out_type[0],
        mesh=mesh,
        scratch_types=scratch,
    )


_SC_CACHE = {}


def _sc_scatter_deg(table, src_p, dst_p):
    if True not in _SC_CACHE:
        _SC_CACHE[True] = _make_sc_scatter(True)
    return _SC_CACHE[True](table, src_p, dst_p)


def _sc_scatter(table, src_p, dst_p):
    if False not in _SC_CACHE:
        _SC_CACHE[False] = _make_sc_scatter(False)
    return _SC_CACHE[False](table, src_p, dst_p)


BLK = 512
GRID = (NACC + BLK - 1) // BLK


def _dinv(pd_ref):
    deg = pd_ref[0] + pd_ref[1] + 1.0
    return lax.rsqrt(deg)


def _tc_a_body(p_ref, pd_ref, x_ref, w1_ref, b1_ref, w2_ref, b2_ref,
               wg2_ref, y2_ref):
    agg = p_ref[0] + p_ref[1] + x_ref[...]
    h = jnp.maximum(
        jnp.dot(agg, w1_ref[...], preferred_element_type=jnp.float32)
        + b1_ref[...], 0.0)
    h = jnp.dot(h, w2_ref[...], preferred_element_type=jnp.float32) \
        + b2_ref[...]
    hr = jnp.maximum(h, 0.0)
    t2 = jnp.dot(hr, wg2_ref[...], preferred_element_type=jnp.float32)
    y2_ref[...] = _dinv(pd_ref)[:, None] * t2


def _tc_b_body(p_ref, pd_ref, y2_ref, bg2_ref, wg3_ref, y3_ref):
    dinv = _dinv(pd_ref)[:, None]
    out2 = dinv * (p_ref[0] + p_ref[1] + y2_ref[...]) + bg2_ref[...]
    t3 = jnp.dot(out2, wg3_ref[...], preferred_element_type=jnp.float32)
    y3_ref[...] = dinv * t3


def _tc_c_body(p_ref, pd_ref, y3_ref, bg3_ref, wl_ref, bl_ref,
               lsm_ref, emb_ref):
    dinv = _dinv(pd_ref)[:, None]
    emb = dinv * (p_ref[0] + p_ref[1] + y3_ref[...]) + bg3_ref[...]
    logits = jnp.dot(emb, wl_ref[...], preferred_element_type=jnp.float32) \
        + bl_ref[...]
    m = jnp.max(logits, axis=1, keepdims=True)
    z = logits - m
    lse = jnp.log(jnp.sum(jnp.exp(z), axis=1, keepdims=True))
    lsm_ref[...] = z - lse
    emb_ref[...] = emb


def _row_spec(cols):
    return pl.BlockSpec((BLK, cols), lambda i: (i, 0))


_P_SPEC = pl.BlockSpec((2, BLK, H), lambda i: (0, i, 0))
_PD_SPEC = pl.BlockSpec((2, BLK), lambda i: (0, i))


def _w_spec(r, c):
    return pl.BlockSpec((r, c), lambda i: (0, 0))


_tc_a = pl.pallas_call(
    _tc_a_body,
    grid=(GRID,),
    in_specs=[_P_SPEC, _PD_SPEC, _row_spec(D), _w_spec(D, H), _w_spec(1, H),
              _w_spec(H, H), _w_spec(1, H), _w_spec(H, H)],
    out_specs=_row_spec(H),
    out_shape=jax.ShapeDtypeStruct((N, H), jnp.float32),
)

_tc_b = pl.pallas_call(
    _tc_b_body,
    grid=(GRID,),
    in_specs=[_P_SPEC, _PD_SPEC, _row_spec(H), _w_spec(1, H), _w_spec(H, H)],
    out_specs=_row_spec(H),
    out_shape=jax.ShapeDtypeStruct((N, H), jnp.float32),
)

_tc_c = pl.pallas_call(
    _tc_c_body,
    grid=(GRID,),
    in_specs=[_P_SPEC, _PD_SPEC, _row_spec(H), _w_spec(1, H), _w_spec(H, O),
              _w_spec(1, O)],
    out_specs=(_row_spec(O), _row_spec(H)),
    out_shape=(jax.ShapeDtypeStruct((N, O), jnp.float32),
               jax.ShapeDtypeStruct((N, H), jnp.float32)),
)


def kernel(x, edge_index, W1, b1, W2, b2, Wg2, bg2, Wg3, bg3, Wl, bl):
    src = edge_index[0].astype(jnp.int32)
    dst = edge_index[1].astype(jnp.int32)
    npad = E_PAD - E
    pad_iota = lax.iota(jnp.int32, npad)
    src_p = jnp.concatenate(
        [src, pad_iota % N]).reshape(E_PAD // CHUNK, CHUNK)
    dst_p = jnp.concatenate(
        [dst, N + pad_iota % (NACC - N)]
    ).reshape(E_PAD // (CHUNK // 2), CHUNK // 2)

    b1r = b1.reshape(1, H)
    b2r = b2.reshape(1, H)
    bg2r = bg2.reshape(1, H)
    bg3r = bg3.reshape(1, H)
    blr = bl.reshape(1, O)

    p1, pdeg = _sc_scatter_deg(x, src_p, dst_p)
    y2 = _tc_a(p1, pdeg, x, W1, b1r, W2, b2r, Wg2)
    p2 = _sc_scatter(y2, src_p, dst_p)
    y3 = _tc_b(p2, pdeg, y2, bg2r, Wg3)
    p3 = _sc_scatter(y3, src_p, dst_p)
    lsm, emb = _tc_c(p3, pdeg, y3, bg3r, Wl, blr)
    return (lsm, emb)


# speedup vs baseline: 21.3565x; 1.0251x over previous
import functools

import jax
import jax.numpy as jnp
from jax import lax
from jax.experimental import pallas as pl
from jax.experimental.pallas import tpu as pltpu
from jax.experimental.pallas import tpu_sc as plsc

N = 10000
E = 320000
D = 128
H = 128
O = 64

NC = 2
NS = 16
NW = NC * NS

CHUNK = 128
CPT = 80
EPT = CPT * CHUNK
E_PAD = EPT * NW
NACC = 10240
RPT = NACC // NS
IDXB = 16
DUMMY_DST = N


def _sc_scatter_body(with_deg, *refs):
    if with_deg:
        (table, src2d, dst2d, out, deg_out,
         src_v0, src_v1, dst_v0, dst_v1, buf_a, buf_b, acc,
         sems_a, sems_b, sems_sa, sems_sb, sems_ix,
         ones_v, degs_v, dacc) = refs
    else:
        (table, src2d, dst2d, out,
         src_v0, src_v1, dst_v0, dst_v1, buf_a, buf_b, acc,
         sems_a, sems_b, sems_sa, sems_sb, sems_ix) = refs

    c = lax.axis_index("c")
    s = lax.axis_index("s")
    wid = s * NC + c

    def zfill_body(r, carry):
        for cc in range(H // 16):
            buf_b[r, pl.ds(cc * 16, 16)] = jnp.zeros((16,), jnp.float32)
        return carry
    lax.fori_loop(0, CHUNK, zfill_body, 0)

    def zero_body(i, carry):
        pltpu.sync_copy(buf_b, acc.at[pl.ds(s * RPT + i * CHUNK, CHUNK)])
        return carry
    lax.fori_loop(0, RPT // CHUNK, zero_body, 0)

    if with_deg:
        for cc in range(CHUNK // 2 // 16):
            ones_v[pl.ds(cc * 16, 16)] = jnp.ones((16,), jnp.float32)

        def dzero_body(i, carry):
            degs_v[pl.ds(i * 16, 16)] = jnp.zeros((16,), jnp.float32)
            return carry
        lax.fori_loop(0, RPT // 16, dzero_body, 0)
        pltpu.sync_copy(degs_v, dacc.at[pl.ds(s * RPT, RPT)])

    plsc.subcore_barrier()

    QTR = CHUNK // 4

    def gather(src_v, j, buf, sems):
        for q in range(4):
            pltpu.async_copy(table.at[src_v.at[j, pl.ds(q * QTR, QTR)]],
                             buf.at[pl.ds(q * QTR, QTR)], sems[q])

    def wait_gather(src_v, j, buf, sems):
        for q in range(4):
            pltpu.make_async_copy(table.at[src_v.at[j, pl.ds(q * QTR, QTR)]],
                                  buf.at[pl.ds(q * QTR, QTR)],
                                  sems[q]).wait()

    HALF = CHUNK // 2

    def scatter(dst_v, j, buf, sems):
        pltpu.async_copy(buf.at[pl.ds(0, HALF)], acc.at[dst_v.at[2 * j]],
                         sems[0], add=True)
        pltpu.async_copy(buf.at[pl.ds(HALF, HALF)],
                         acc.at[dst_v.at[2 * j + 1]], sems[1], add=True)

    def wait_scatter(dst_v, j, buf, sems):
        pltpu.make_async_copy(buf.at[pl.ds(0, HALF)],
                              acc.at[dst_v.at[2 * j]], sems[0]).wait()
        pltpu.make_async_copy(buf.at[pl.ds(HALF, HALF)],
                              acc.at[dst_v.at[2 * j + 1]], sems[1]).wait()

    def deg_scatter(dst_v, j):
        pltpu.sync_copy(ones_v, dacc.at[dst_v.at[2 * j]], add=True)
        pltpu.sync_copy(ones_v, dacc.at[dst_v.at[2 * j + 1]], add=True)

    def stage_idx(b, src_v, dst_v, sync):
        sc = src2d.at[pl.ds(wid * CPT + b * IDXB, IDXB)]
        dc = dst2d.at[pl.ds(2 * (wid * CPT + b * IDXB), 2 * IDXB)]
        if sync:
            pltpu.sync_copy(sc, src_v)
            pltpu.sync_copy(dc, dst_v)
        else:
            pltpu.async_copy(sc, src_v, sems_ix[0])
            pltpu.async_copy(dc, dst_v, sems_ix[1])

    def wait_idx(b, src_v, dst_v):
        sc = src2d.at[pl.ds(wid * CPT + b * IDXB, IDXB)]
        dc = dst2d.at[pl.ds(2 * (wid * CPT + b * IDXB), 2 * IDXB)]
        pltpu.make_async_copy(sc, src_v, sems_ix[0]).wait()
        pltpu.make_async_copy(dc, dst_v, sems_ix[1]).wait()

    NBLK = CPT // IDXB
    idx_bufs = ((src_v0, dst_v0), (src_v1, dst_v1))
    stage_idx(0, src_v0, dst_v0, True)
    for b in range(NBLK):
        src_v, dst_v = idx_bufs[b % 2]
        if b + 1 < NBLK:
            stage_idx(b + 1, *idx_bufs[(b + 1) % 2], False)
        gather(src_v, 0, buf_a, sems_a)
        gather(src_v, 1, buf_b, sems_b)

        def body(i, carry2, src_v=src_v, dst_v=dst_v):
            j0 = 2 * i
            j1 = 2 * i + 1
            wait_gather(src_v, j0, buf_a, sems_a)
            scatter(dst_v, j0, buf_a, sems_sa)
            if with_deg:
                deg_scatter(dst_v, j0)
            wait_gather(src_v, j1, buf_b, sems_b)
            scatter(dst_v, j1, buf_b, sems_sb)
            if with_deg:
                deg_scatter(dst_v, j1)

            @pl.when(i < IDXB // 2 - 1)
            def _refill():
                wait_scatter(dst_v, j0, buf_a, sems_sa)
                gather(src_v, j0 + 2, buf_a, sems_a)
                wait_scatter(dst_v, j1, buf_b, sems_sb)
                gather(src_v, j1 + 2, buf_b, sems_b)
            return carry2
        lax.fori_loop(0, IDXB // 2, body, 0)
        wait_scatter(dst_v, IDXB - 2, buf_a, sems_sa)
        wait_scatter(dst_v, IDXB - 1, buf_b, sems_sb)
        if b + 1 < NBLK:
            wait_idx(b + 1, *idx_bufs[(b + 1) % 2])

    plsc.subcore_barrier()

    pending = [None, None]
    for k in range(RPT // CHUNK):
        r0 = s * RPT + k * CHUNK
        buf, sem = (buf_a, sems_sa[0]) if k % 2 == 0 else (buf_b, sems_sb[0])
        if pending[k % 2] is not None:
            pending[k % 2].wait()
        pltpu.sync_copy(acc.at[pl.ds(r0, CHUNK)], buf)
        pending[k % 2] = pltpu.async_copy(buf, out.at[c, pl.ds(r0, CHUNK)],
                                          sem)
    for p in pending:
        if p is not None:
            p.wait()

    if with_deg:
        pltpu.sync_copy(dacc.at[pl.ds(s * RPT, RPT)], degs_v)
        pltpu.sync_copy(degs_v, deg_out.at[c, pl.ds(s * RPT, RPT)])


def _make_sc_scatter(with_deg):
    out_type = [jax.ShapeDtypeStruct((NC, NACC, H), jnp.float32)]
    if with_deg:
        out_type.append(jax.ShapeDtypeStruct((NC, NACC), jnp.float32))
    scratch = [
        pltpu.VMEM((IDXB, CHUNK), jnp.int32),
        pltpu.VMEM((IDXB, CHUNK), jnp.int32),
        pltpu.VMEM((2 * IDXB, CHUNK // 2), jnp.int32),
        pltpu.VMEM((2 * IDXB, CHUNK // 2), jnp.int32),
        pltpu.VMEM((CHUNK, H), jnp.float32),
        pltpu.VMEM((CHUNK, H), jnp.float32),
        pltpu.VMEM_SHARED((NACC, H), jnp.float32),
        [pltpu.SemaphoreType.DMA] * 4,
        [pltpu.SemaphoreType.DMA] * 4,
        [pltpu.SemaphoreType.DMA] * 2,
        [pltpu.SemaphoreType.DMA] * 2,
        [pltpu.SemaphoreType.DMA] * 2,
    ]
    if with_deg:
        scratch += [
            pltpu.VMEM((CHUNK // 2,), jnp.float32),
            pltpu.VMEM((RPT,), jnp.float32),
            pltpu.VMEM_SHARED((NACC,), jnp.float32),
        ]
    mesh = plsc.VectorSubcoreMesh(core_axis_name="c", subcore_axis_name="s",
                                  num_cores=NC, num_subcores=NS)
    return pl.kernel(
        functools.partial(_sc_scatter_body, with_deg),
        out_type=tuple(out_type) if with_deg else out_type[0],
        mesh=mesh,
        scratch_types=scratch,
    )


_SC_CACHE = {}


def _sc_scatter_deg(table, src_p, dst_p):
    if True not in _SC_CACHE:
        _SC_CACHE[True] = _make_sc_scatter(True)
    return _SC_CACHE[True](table, src_p, dst_p)


def _sc_scatter(table, src_p, dst_p):
    if False not in _SC_CACHE:
        _SC_CACHE[False] = _make_sc_scatter(False)
    return _SC_CACHE[False](table, src_p, dst_p)


BLK = 512
GRID = (NACC + BLK - 1) // BLK


def _dinv(pd_ref):
    deg = pd_ref[0] + pd_ref[1] + 1.0
    return lax.rsqrt(deg)


def _tc_a_body(p_ref, pd_ref, x_ref, w1_ref, b1_ref, w2_ref, b2_ref,
               wg2_ref, y2_ref):
    agg = p_ref[0] + p_ref[1] + x_ref[...]
    h = jnp.maximum(
        jnp.dot(agg, w1_ref[...], preferred_element_type=jnp.float32)
        + b1_ref[...], 0.0)
    h = jnp.dot(h, w2_ref[...], preferred_element_type=jnp.float32) \
        + b2_ref[...]
    hr = jnp.maximum(h, 0.0)
    t2 = jnp.dot(hr, wg2_ref[...], preferred_element_type=jnp.float32)
    y2_ref[...] = _dinv(pd_ref)[:, None] * t2


def _tc_b_body(p_ref, pd_ref, y2_ref, bg2_ref, wg3_ref, y3_ref):
    dinv = _dinv(pd_ref)[:, None]
    out2 = dinv * (p_ref[0] + p_ref[1] + y2_ref[...]) + bg2_ref[...]
    t3 = jnp.dot(out2, wg3_ref[...], preferred_element_type=jnp.float32)
    y3_ref[...] = dinv * t3


def _tc_c_body(p_ref, pd_ref, y3_ref, bg3_ref, wl_ref, bl_ref,
               lsm_ref, emb_ref):
    dinv = _dinv(pd_ref)[:, None]
    emb = dinv * (p_ref[0] + p_ref[1] + y3_ref[...]) + bg3_ref[...]
    logits = jnp.dot(emb, wl_ref[...], preferred_element_type=jnp.float32) \
        + bl_ref[...]
    m = jnp.max(logits, axis=1, keepdims=True)
    z = logits - m
    lse = jnp.log(jnp.sum(jnp.exp(z), axis=1, keepdims=True))
    lsm_ref[...] = z - lse
    emb_ref[...] = emb


def _row_spec(cols):
    return pl.BlockSpec((BLK, cols), lambda i: (i, 0))


_P_SPEC = pl.BlockSpec((2, BLK, H), lambda i: (0, i, 0))
_PD_SPEC = pl.BlockSpec((2, BLK), lambda i: (0, i))


def _w_spec(r, c):
    return pl.BlockSpec((r, c), lambda i: (0, 0))


_tc_a = pl.pallas_call(
    _tc_a_body,
    grid=(GRID,),
    in_specs=[_P_SPEC, _PD_SPEC, _row_spec(D), _w_spec(D, H), _w_spec(1, H),
              _w_spec(H, H), _w_spec(1, H), _w_spec(H, H)],
    out_specs=_row_spec(H),
    out_shape=jax.ShapeDtypeStruct((N, H), jnp.float32),
)

_tc_b = pl.pallas_call(
    _tc_b_body,
    grid=(GRID,),
    in_specs=[_P_SPEC, _PD_SPEC, _row_spec(H), _w_spec(1, H), _w_spec(H, H)],
    out_specs=_row_spec(H),
    out_shape=jax.ShapeDtypeStruct((N, H), jnp.float32),
)

_tc_c = pl.pallas_call(
    _tc_c_body,
    grid=(GRID,),
    in_specs=[_P_SPEC, _PD_SPEC, _row_spec(H), _w_spec(1, H), _w_spec(H, O),
              _w_spec(1, O)],
    out_specs=(_row_spec(O), _row_spec(H)),
    out_shape=(jax.ShapeDtypeStruct((N, O), jnp.float32),
               jax.ShapeDtypeStruct((N, H), jnp.float32)),
)


def kernel(x, edge_index, W1, b1, W2, b2, Wg2, bg2, Wg3, bg3, Wl, bl):
    src = edge_index[0].astype(jnp.int32)
    dst = edge_index[1].astype(jnp.int32)
    npad = E_PAD - E
    pad_iota = lax.iota(jnp.int32, npad)
    src_p = jnp.concatenate(
        [src, pad_iota % N]).reshape(E_PAD // CHUNK, CHUNK)
    dst_p = jnp.concatenate(
        [dst, N + pad_iota % (NACC - N)]
    ).reshape(E_PAD // (CHUNK // 2), CHUNK // 2)

    b1r = b1.reshape(1, H)
    b2r = b2.reshape(1, H)
    bg2r = bg2.reshape(1, H)
    bg3r = bg3.reshape(1, H)
    blr = bl.reshape(1, O)

    p1, pdeg = _sc_scatter_deg(x, src_p, dst_p)
    y2 = _tc_a(p1, pdeg, x, W1, b1r, W2, b2r, Wg2)
    p2 = _sc_scatter(y2, src_p, dst_p)
    y3 = _tc_b(p2, pdeg, y2, bg2r, Wg3)
    p3 = _sc_scatter(y3, src_p, dst_p)
    lsm, emb = _tc_c(p3, pdeg, y3, bg3r, Wl, blr)
    return (lsm, emb)


# speedup vs baseline: 22.1384x; 1.0366x over previous
import functools

import jax
import jax.numpy as jnp
from jax import lax
from jax.experimental import pallas as pl
from jax.experimental.pallas import tpu as pltpu
from jax.experimental.pallas import tpu_sc as plsc

N = 10000
E = 320000
D = 128
H = 128
O = 64

NC = 2
NS = 16
NW = NC * NS

CHUNK = 128
CPT = 80
EPT = CPT * CHUNK
E_PAD = EPT * NW
NACC = 10240
RPT = NACC // NS
IDXB = 16
DUMMY_DST = N


def _sc_scatter_body(with_deg, *refs):
    if with_deg:
        (table, src2d, dst2d, out, deg_out,
         src_v0, src_v1, dst_v0, dst_v1, buf_a, buf_b, acc,
         sems_a, sems_b, sems_sa, sems_sb, sems_ix,
         ones_v, degs_v, dacc) = refs
    else:
        (table, src2d, dst2d, out,
         src_v0, src_v1, dst_v0, dst_v1, buf_a, buf_b, acc,
         sems_a, sems_b, sems_sa, sems_sb, sems_ix) = refs

    c = lax.axis_index("c")
    s = lax.axis_index("s")
    wid = s * NC + c

    def zfill_body(r, carry):
        for cc in range(H // 16):
            buf_b[r, pl.ds(cc * 16, 16)] = jnp.zeros((16,), jnp.float32)
        return carry
    lax.fori_loop(0, CHUNK, zfill_body, 0)

    def zero_body(i, carry):
        pltpu.sync_copy(buf_b, acc.at[pl.ds(s * RPT + i * CHUNK, CHUNK)])
        return carry
    lax.fori_loop(0, RPT // CHUNK, zero_body, 0)

    if with_deg:
        for cc in range(CHUNK // 2 // 16):
            ones_v[pl.ds(cc * 16, 16)] = jnp.ones((16,), jnp.float32)

        def dzero_body(i, carry):
            degs_v[pl.ds(i * 16, 16)] = jnp.zeros((16,), jnp.float32)
            return carry
        lax.fori_loop(0, RPT // 16, dzero_body, 0)
        pltpu.sync_copy(degs_v, dacc.at[pl.ds(s * RPT, RPT)])

    plsc.subcore_barrier()

    QTR = CHUNK // 4

    def gather(src_v, j, buf, sems):
        for q in range(4):
            pltpu.async_copy(table.at[src_v.at[j, pl.ds(q * QTR, QTR)]],
                             buf.at[pl.ds(q * QTR, QTR)], sems[q])

    def wait_gather(src_v, j, buf, sems):
        for q in range(4):
            pltpu.make_async_copy(table.at[src_v.at[j, pl.ds(q * QTR, QTR)]],
                                  buf.at[pl.ds(q * QTR, QTR)],
                                  sems[q]).wait()

    HALF = CHUNK // 2

    def scatter(dst_v, j, buf, sems):
        pltpu.async_copy(buf.at[pl.ds(0, HALF)], acc.at[dst_v.at[2 * j]],
                         sems[0], add=True)
        pltpu.async_copy(buf.at[pl.ds(HALF, HALF)],
                         acc.at[dst_v.at[2 * j + 1]], sems[1], add=True)

    def wait_scatter(dst_v, j, buf, sems):
        pltpu.make_async_copy(buf.at[pl.ds(0, HALF)],
                              acc.at[dst_v.at[2 * j]], sems[0]).wait()
        pltpu.make_async_copy(buf.at[pl.ds(HALF, HALF)],
                              acc.at[dst_v.at[2 * j + 1]], sems[1]).wait()

    def deg_scatter(dst_v, j):
        pltpu.sync_copy(ones_v, dacc.at[dst_v.at[2 * j]], add=True)
        pltpu.sync_copy(ones_v, dacc.at[dst_v.at[2 * j + 1]], add=True)

    def stage_idx(b, src_v, dst_v, sync):
        sc = src2d.at[pl.ds(wid * CPT + b * IDXB, IDXB)]
        dc = dst2d.at[pl.ds(2 * (wid * CPT + b * IDXB), 2 * IDXB)]
        if sync:
            pltpu.sync_copy(sc, src_v)
            pltpu.sync_copy(dc, dst_v)
        else:
            pltpu.async_copy(sc, src_v, sems_ix[0])
            pltpu.async_copy(dc, dst_v, sems_ix[1])

    def wait_idx(b, src_v, dst_v):
        sc = src2d.at[pl.ds(wid * CPT + b * IDXB, IDXB)]
        dc = dst2d.at[pl.ds(2 * (wid * CPT + b * IDXB), 2 * IDXB)]
        pltpu.make_async_copy(sc, src_v, sems_ix[0]).wait()
        pltpu.make_async_copy(dc, dst_v, sems_ix[1]).wait()

    NBLK = CPT // IDXB
    idx_bufs = ((src_v0, dst_v0), (src_v1, dst_v1))
    stage_idx(0, src_v0, dst_v0, True)
    for b in range(NBLK):
        src_v, dst_v = idx_bufs[b % 2]
        if b + 1 < NBLK:
            stage_idx(b + 1, *idx_bufs[(b + 1) % 2], False)
        gather(src_v, 0, buf_a, sems_a)
        gather(src_v, 1, buf_b, sems_b)

        def body(i, carry2, src_v=src_v, dst_v=dst_v):
            j0 = 2 * i
            j1 = 2 * i + 1
            wait_gather(src_v, j0, buf_a, sems_a)
            scatter(dst_v, j0, buf_a, sems_sa)
            if with_deg:
                deg_scatter(dst_v, j0)
            wait_gather(src_v, j1, buf_b, sems_b)
            scatter(dst_v, j1, buf_b, sems_sb)
            if with_deg:
                deg_scatter(dst_v, j1)

            @pl.when(i < IDXB // 2 - 1)
            def _refill():
                wait_scatter(dst_v, j0, buf_a, sems_sa)
                gather(src_v, j0 + 2, buf_a, sems_a)
                wait_scatter(dst_v, j1, buf_b, sems_sb)
                gather(src_v, j1 + 2, buf_b, sems_b)
            return carry2
        lax.fori_loop(0, IDXB // 2, body, 0)
        wait_scatter(dst_v, IDXB - 2, buf_a, sems_sa)
        wait_scatter(dst_v, IDXB - 1, buf_b, sems_sb)
        if b + 1 < NBLK:
            wait_idx(b + 1, *idx_bufs[(b + 1) % 2])

    plsc.subcore_barrier()

    pending = [None, None]
    for k in range(RPT // CHUNK):
        r0 = s * RPT + k * CHUNK
        buf, sem = (buf_a, sems_sa[0]) if k % 2 == 0 else (buf_b, sems_sb[0])
        if pending[k % 2] is not None:
            pending[k % 2].wait()
        pltpu.sync_copy(acc.at[pl.ds(r0, CHUNK)], buf)
        pending[k % 2] = pltpu.async_copy(buf, out.at[c, pl.ds(r0, CHUNK)],
                                          sem)
    for p in pending:
        if p is not None:
            p.wait()

    if with_deg:
        pltpu.sync_copy(dacc.at[pl.ds(s * RPT, RPT)], degs_v)
        pltpu.sync_copy(degs_v, deg_out.at[c, pl.ds(s * RPT, RPT)])


def _make_sc_scatter(with_deg):
    out_type = [jax.ShapeDtypeStruct((NC, NACC, H), jnp.float32)]
    if with_deg:
        out_type.append(jax.ShapeDtypeStruct((NC, NACC), jnp.float32))
    scratch = [
        pltpu.VMEM((IDXB, CHUNK), jnp.int32),
        pltpu.VMEM((IDXB, CHUNK), jnp.int32),
        pltpu.VMEM((2 * IDXB, CHUNK // 2), jnp.int32),
        pltpu.VMEM((2 * IDXB, CHUNK // 2), jnp.int32),
        pltpu.VMEM((CHUNK, H), jnp.float32),
        pltpu.VMEM((CHUNK, H), jnp.float32),
        pltpu.VMEM_SHARED((NACC, H), jnp.float32),
        [pltpu.SemaphoreType.DMA] * 4,
        [pltpu.SemaphoreType.DMA] * 4,
        [pltpu.SemaphoreType.DMA] * 2,
        [pltpu.SemaphoreType.DMA] * 2,
        [pltpu.SemaphoreType.DMA] * 2,
    ]
    if with_deg:
        scratch += [
            pltpu.VMEM((CHUNK // 2,), jnp.float32),
            pltpu.VMEM((RPT,), jnp.float32),
            pltpu.VMEM_SHARED((NACC,), jnp.float32),
        ]
    mesh = plsc.VectorSubcoreMesh(core_axis_name="c", subcore_axis_name="s",
                                  num_cores=NC, num_subcores=NS)
    return pl.kernel(
        functools.partial(_sc_scatter_body, with_deg),
        out_type=tuple(out_type) if with_deg else out_type[0],
        mesh=mesh,
        scratch_types=scratch,
    )


_SC_CACHE = {}


def _sc_scatter_deg(table, src_p, dst_p):
    if True not in _SC_CACHE:
        _SC_CACHE[True] = _make_sc_scatter(True)
    return _SC_CACHE[True](table, src_p, dst_p)


def _sc_scatter(table, src_p, dst_p):
    if False not in _SC_CACHE:
        _SC_CACHE[False] = _make_sc_scatter(False)
    return _SC_CACHE[False](table, src_p, dst_p)


BLK = 1024
GRID = (NACC + BLK - 1) // BLK


def _dinv(pd_ref):
    deg = pd_ref[0] + pd_ref[1] + 1.0
    return lax.rsqrt(deg)


def _tc_a_body(p_ref, pd_ref, x_ref, w1_ref, b1_ref, w2_ref, b2_ref,
               wg2_ref, y2_ref):
    agg = p_ref[0] + p_ref[1] + x_ref[...]
    h = jnp.maximum(
        jnp.dot(agg, w1_ref[...], preferred_element_type=jnp.float32)
        + b1_ref[...], 0.0)
    h = jnp.dot(h, w2_ref[...], preferred_element_type=jnp.float32) \
        + b2_ref[...]
    hr = jnp.maximum(h, 0.0)
    t2 = jnp.dot(hr, wg2_ref[...], preferred_element_type=jnp.float32)
    y2_ref[...] = _dinv(pd_ref)[:, None] * t2


def _tc_b_body(p_ref, pd_ref, y2_ref, bg2_ref, wg3_ref, y3_ref):
    dinv = _dinv(pd_ref)[:, None]
    out2 = dinv * (p_ref[0] + p_ref[1] + y2_ref[...]) + bg2_ref[...]
    t3 = jnp.dot(out2, wg3_ref[...], preferred_element_type=jnp.float32)
    y3_ref[...] = dinv * t3


def _tc_c_body(p_ref, pd_ref, y3_ref, bg3_ref, wl_ref, bl_ref,
               lsm_ref, emb_ref):
    dinv = _dinv(pd_ref)[:, None]
    emb = dinv * (p_ref[0] + p_ref[1] + y3_ref[...]) + bg3_ref[...]
    logits = jnp.dot(emb, wl_ref[...], preferred_element_type=jnp.float32) \
        + bl_ref[...]
    m = jnp.max(logits, axis=1, keepdims=True)
    z = logits - m
    lse = jnp.log(jnp.sum(jnp.exp(z), axis=1, keepdims=True))
    lsm_ref[...] = z - lse
    emb_ref[...] = emb


def _row_spec(cols):
    return pl.BlockSpec((BLK, cols), lambda i: (i, 0))


_P_SPEC = pl.BlockSpec((2, BLK, H), lambda i: (0, i, 0))
_PD_SPEC = pl.BlockSpec((2, BLK), lambda i: (0, i))


def _w_spec(r, c):
    return pl.BlockSpec((r, c), lambda i: (0, 0))


_tc_a = pl.pallas_call(
    _tc_a_body,
    grid=(GRID,),
    in_specs=[_P_SPEC, _PD_SPEC, _row_spec(D), _w_spec(D, H), _w_spec(1, H),
              _w_spec(H, H), _w_spec(1, H), _w_spec(H, H)],
    out_specs=_row_spec(H),
    out_shape=jax.ShapeDtypeStruct((N, H), jnp.float32),
)

_tc_b = pl.pallas_call(
    _tc_b_body,
    grid=(GRID,),
    in_specs=[_P_SPEC, _PD_SPEC, _row_spec(H), _w_spec(1, H), _w_spec(H, H)],
    out_specs=_row_spec(H),
    out_shape=jax.ShapeDtypeStruct((N, H), jnp.float32),
)

_tc_c = pl.pallas_call(
    _tc_c_body,
    grid=(GRID,),
    in_specs=[_P_SPEC, _PD_SPEC, _row_spec(H), _w_spec(1, H), _w_spec(H, O),
              _w_spec(1, O)],
    out_specs=(_row_spec(O), _row_spec(H)),
    out_shape=(jax.ShapeDtypeStruct((N, O), jnp.float32),
               jax.ShapeDtypeStruct((N, H), jnp.float32)),
)


def kernel(x, edge_index, W1, b1, W2, b2, Wg2, bg2, Wg3, bg3, Wl, bl):
    src = edge_index[0].astype(jnp.int32)
    dst = edge_index[1].astype(jnp.int32)
    npad = E_PAD - E
    pad_iota = lax.iota(jnp.int32, npad)
    src_p = jnp.concatenate(
        [src, pad_iota % N]).reshape(E_PAD // CHUNK, CHUNK)
    dst_p = jnp.concatenate(
        [dst, N + pad_iota % (NACC - N)]
    ).reshape(E_PAD // (CHUNK // 2), CHUNK // 2)

    b1r = b1.reshape(1, H)
    b2r = b2.reshape(1, H)
    bg2r = bg2.reshape(1, H)
    bg3r = bg3.reshape(1, H)
    blr = bl.reshape(1, O)

    p1, pdeg = _sc_scatter_deg(x, src_p, dst_p)
    y2 = _tc_a(p1, pdeg, x, W1, b1r, W2, b2r, Wg2)
    p2 = _sc_scatter(y2, src_p, dst_p)
    y3 = _tc_b(p2, pdeg, y2, bg2r, Wg3)
    p3 = _sc_scatter(y3, src_p, dst_p)
    lsm, emb = _tc_c(p3, pdeg, y3, bg3r, Wl, blr)
    return (lsm, emb)
